# baseline (speedup 1.0000x reference)
import jax
import jax.numpy as jnp
from jax import lax
from jax.experimental import pallas as pl
from jax.experimental.pallas import tpu as pltpu
from jax.experimental.pallas import tpu_sc as plsc

_C = 128
_P = 7
_CELLS = 352
_GPB = _CELLS // 16
_NBOX = 1024
_NW = 32
_BPW = _NBOX // _NW
_GPW = _BPW * _GPB
_TBL_ROWS = 37504


def _meta_body(boxes_ref, idx_ref, lz_ref, ly_ref, lx_ref):
    b = boxes_ref[...]
    x1 = b[:, 0:1]
    y1 = b[:, 1:2]
    z1 = b[:, 2:3]
    x2 = b[:, 3:4]
    y2 = b[:, 4:5]
    z2 = b[:, 5:6]
    vol = (x2 - x1) * (y2 - y1) * (z2 - z1)
    s = jnp.power(jnp.maximum(vol, 0.0), 1.0 / 3.0)
    tgt = jnp.floor(4.0 + jnp.log2(s / 224.0) + 1e-6)
    lvl = jnp.clip(tgt, 2.0, 5.0) - 2.0

    def sel(c0, c1, c2, c3):
        return jnp.where(
            lvl < 0.5, c0, jnp.where(lvl < 1.5, c1, jnp.where(lvl < 2.5, c2, c3)))

    scale = sel(0.25, 0.125, 0.0625, 0.03125)
    dimf = sel(32.0, 16.0, 8.0, 4.0)
    basef = sel(0.0, 32768.0, 36864.0, 37376.0)
    x1s = x1 * scale
    y1s = y1 * scale
    z1s = z1 * scale
    rw = jnp.maximum(x2 * scale - x1s, 1.0) * (1.0 / _P)
    rh = jnp.maximum(y2 * scale - y1s, 1.0) * (1.0 / _P)
    rd = jnp.maximum(z2 * scale - z1s, 1.0) * (1.0 / _P)
    cell = lax.broadcasted_iota(jnp.int32, (1, _CELLS), 1)
    pxf = (cell % _P).astype(jnp.float32)
    pyf = ((cell // _P) % _P).astype(jnp.float32)
    pzf = (cell // (_P * _P)).astype(jnp.float32)
    mx = dimf - 1.0
    x = jnp.clip(x1s + (pxf + 0.5) * rw, 0.0, mx)
    y = jnp.clip(y1s + (pyf + 0.5) * rh, 0.0, mx)
    z = jnp.clip(z1s + (pzf + 0.5) * rd, 0.0, mx)
    x0 = jnp.floor(x)
    y0 = jnp.floor(y)
    z0 = jnp.floor(z)
    lx_ref[...] = x - x0
    ly_ref[...] = y - y0
    lz_ref[...] = z - z0
    idx_ref[...] = (basef + (z0 * dimf + y0) * dimf + x0
                    + lvl * 1048576.0).astype(jnp.int32)


def _compute_meta(boxes_p):
    f32 = jnp.float32
    i32 = jnp.int32
    outs = pl.pallas_call(
        _meta_body,
        out_shape=[
            jax.ShapeDtypeStruct((_NBOX, _CELLS), i32),
            jax.ShapeDtypeStruct((_NBOX, _CELLS), f32),
            jax.ShapeDtypeStruct((_NBOX, _CELLS), f32),
            jax.ShapeDtypeStruct((_NBOX, _CELLS), f32),
        ],
    )(boxes_p)
    return [o.reshape(-1) for o in outs]


def _sc_body(tbl, midx, mlz, mly, mlx, out,
             vidx, vlz, vly, vlx, vl0, vl1, vl2, vl3, vr0, vr1, vr2, vr3,
             vo0, vo1, sm0, sm1, sm2, sm3, om0, om1):
    wid = lax.axis_index("s") * 2 + lax.axis_index("c")
    e0 = wid * (_GPW * 16)
    for src, dst in ((midx, vidx), (mlz, vlz), (mly, vly), (mlx, vlx)):
        pltpu.sync_copy(src.at[pl.ds(e0, _GPW * 16)], dst)

    lane = lax.broadcasted_iota(jnp.int32, (16,), 0)
    zeros = jnp.zeros((16,), jnp.int32)
    ones = zeros + 1
    n0 = wid * _BPW
    vls = (vl0, vl1, vl2, vl3)
    vrs = (vr0, vr1, vr2, vr3)
    sms = (sm0, sm1, sm2, sm3)
    vos = (vo0, vo1)
    oms = (om0, om1)
    dn = lax.GatherDimensionNumbers(
        offset_dims=(), collapsed_slice_dims=(0,), start_index_map=(0,))

    def fire(gl, s):
        gv = zeros + gl * 16 + lane
        ivp = plsc.load_gather(vidx, [gv])
        lvl = lax.shift_right_logical(ivp, 20)
        iv = ivp - lax.shift_left(lvl, 20)
        expw = 5 - lvl
        wv = lax.shift_left(ones, expw)
        hwv = lax.shift_left(ones, expw + expw)
        k = 0
        for az in range(2):
            for ay in range(2):
                gvox = iv
                if az:
                    gvox = gvox + hwv
                if ay:
                    gvox = gvox + wv
                par = gvox & 1
                ik = lax.shift_right_logical(gvox, 1) + par * (_TBL_ROWS // 2)
                vls[s][pl.ds(k * 16, 16)] = ik
                k += 1
        pltpu.async_copy(tbl.at[vls[s]], vrs[s], sms[s])

    fire(jnp.int32(0), 0)
    fire(jnp.int32(1), 1)
    fire(jnp.int32(2), 2)

    def box_loop(bi, carry):
        nb0 = n0 + bi * 2
        g0 = bi * (2 * _GPB)

        def quad_loop(it, carry2):
            for half in range(4):
                s = half
                sub2 = it * 4 + half
                g = g0 + sub2
                n = nb0 + jnp.where(sub2 >= _GPB, 1, 0)
                sub = jnp.where(sub2 >= _GPB, sub2 - _GPB, sub2)
                fire(jnp.minimum(g + 3, _GPW - 1), (half + 3) % 4)
                pltpu.make_async_copy(tbl.at[vls[s]], vrs[s], sms[s]).wait()
                gv = zeros + g * 16 + lane
                tz = plsc.load_gather(vlz, [gv])
                ty = plsc.load_gather(vly, [gv])
                tx = plsc.load_gather(vlx, [gv])
                wz = (1.0 - tz, tz)
                wy = (1.0 - ty, ty)
                wx = (1.0 - tx, tx)
                wk = [wz[az] * wy[ay] * wx[ax]
                      for az in range(2) for ay in range(2) for ax in range(2)]
                vr = vrs[s]
                so = half % 2
                vo = vos[so]

                @pl.when(g >= 2)
                def _(vo=vo, so=so):
                    pltpu.make_async_copy(
                        vo, out.at[0, pl.ds(0, 16), :], oms[so]).wait()

                def p_loop(p, carry4, wk=wk, vr=vr, vo=vo):
                    pv = zeros + p
                    wb = [lax.gather(wk[k2], pv[:, None], dn, (1,),
                                     mode=lax.GatherScatterMode.PROMISE_IN_BOUNDS)
                          for k2 in range(8)]
                    for j in range(4):
                        aa = None
                        ab = None
                        for k4 in range(4):
                            for pos in range(2):
                                vi = vr[k4 * 16 + p,
                                        pl.ds(pos * 64 + j * 16, 16)]
                                v = plsc.bitcast(vi, jnp.bfloat16)
                                ua, ub = plsc.unpack(
                                    v, format=plsc.PackFormat.INTERLEAVED,
                                    preferred_element_type=jnp.float32)
                                w = wb[k4 * 2 + pos]
                                if aa is None:
                                    aa = ua * w
                                    ab = ub * w
                                else:
                                    aa = aa + ua * w
                                    ab = ab + ub * w
                        vo[p, pl.ds(j * 32, 16)] = aa
                        vo[p, pl.ds(j * 32 + 16, 16)] = ab
                    return carry4

                lax.fori_loop(0, 16, p_loop, 0, unroll=2)
                off = pl.multiple_of(sub * 16, 16)
                pltpu.async_copy(vo, out.at[n, pl.ds(off, 16), :], oms[so])
            return carry2

        lax.fori_loop(0, _GPB // 2, quad_loop, 0)
        return carry

    lax.fori_loop(0, _BPW // 2, box_loop, 0)
    pltpu.make_async_copy(tbl.at[vls[0]], vrs[0], sms[0]).wait()
    pltpu.make_async_copy(tbl.at[vls[1]], vrs[1], sms[1]).wait()
    pltpu.make_async_copy(tbl.at[vls[2]], vrs[2], sms[2]).wait()
    pltpu.make_async_copy(vo0, out.at[0, pl.ds(0, 16), :], om0).wait()
    pltpu.make_async_copy(vo1, out.at[0, pl.ds(0, 16), :], om1).wait()


def _sc_align(tbl, meta):
    midx, mlz, mly, mlx = meta
    i32 = jnp.int32
    f32 = jnp.float32
    ne = _GPW * 16
    return pl.kernel(
        _sc_body,
        out_type=jax.ShapeDtypeStruct((_NBOX, _CELLS, _C), f32),
        mesh=plsc.VectorSubcoreMesh(core_axis_name="c", subcore_axis_name="s"),
        compiler_params=pltpu.CompilerParams(
            needs_layout_passes=False, use_tc_tiling_on_sc=False),
        scratch_types=[
            pltpu.VMEM((ne,), i32),
            pltpu.VMEM((ne,), f32),
            pltpu.VMEM((ne,), f32),
            pltpu.VMEM((ne,), f32),
            pltpu.VMEM((64,), i32),
            pltpu.VMEM((64,), i32),
            pltpu.VMEM((64,), i32),
            pltpu.VMEM((64,), i32),
            pltpu.VMEM((64, _C), i32),
            pltpu.VMEM((64, _C), i32),
            pltpu.VMEM((64, _C), i32),
            pltpu.VMEM((64, _C), i32),
            pltpu.VMEM((16, _C), f32),
            pltpu.VMEM((16, _C), f32),
            pltpu.SemaphoreType.DMA,
            pltpu.SemaphoreType.DMA,
            pltpu.SemaphoreType.DMA,
            pltpu.SemaphoreType.DMA,
            pltpu.SemaphoreType.DMA,
            pltpu.SemaphoreType.DMA,
        ],
    )(tbl, midx, mlz, mly, mlx)


_CH_PERM = tuple(
    (p % 2) * 16 + (p % 32) // 2 + (p // 32) * 32 for p in range(_C))


def kernel(feat0, feat1, feat2, feat3, boxes):
    tbl = jnp.concatenate(
        [f[0].reshape(_C, -1).T for f in (feat0, feat1, feat2, feat3)], axis=0)
    tbl = jnp.pad(tbl, ((0, _TBL_ROWS - tbl.shape[0]), (0, 0)))
    tbl = tbl[:, jnp.array(_CH_PERM, dtype=jnp.int32)].astype(jnp.bfloat16)
    tbl = lax.bitcast_convert_type(
        tbl.reshape(_TBL_ROWS, _C // 2, 2), jnp.int32)
    tblp = jnp.pad(tbl, ((0, 2), (0, 0)))
    ta = tblp[:_TBL_ROWS].reshape(_TBL_ROWS // 2, _C)
    tb = tblp[1:_TBL_ROWS + 1].reshape(_TBL_ROWS // 2, _C)
    tbl = jnp.concatenate([ta, tb], axis=0)
    nb = boxes.shape[0]
    boxes_p = jnp.pad(boxes, ((0, _NBOX - nb), (0, 0)))
    meta = _compute_meta(boxes_p)
    out = _sc_align(tbl, meta)
    return out[:nb, :343, :].transpose(0, 2, 1).reshape(nb, _C, _P, _P, _P)


# speedup vs baseline: 1.0536x; 1.0536x over previous
import jax
import jax.numpy as jnp
from jax import lax
from jax.experimental import pallas as pl
from jax.experimental.pallas import tpu as pltpu
from jax.experimental.pallas import tpu_sc as plsc

_C = 128
_P = 7
_CELLS = 352
_GPB = _CELLS // 16
_NBOX = 1024
_NW = 32
_BPW = _NBOX // _NW
_GPW = _BPW * _GPB
_TBL_ROWS = 37504


def _meta_body(boxes_ref, idx_ref, lz_ref, ly_ref, lx_ref):
    b = boxes_ref[...]
    x1 = b[:, 0:1]
    y1 = b[:, 1:2]
    z1 = b[:, 2:3]
    x2 = b[:, 3:4]
    y2 = b[:, 4:5]
    z2 = b[:, 5:6]
    vol = (x2 - x1) * (y2 - y1) * (z2 - z1)
    s = jnp.power(jnp.maximum(vol, 0.0), 1.0 / 3.0)
    tgt = jnp.floor(4.0 + jnp.log2(s / 224.0) + 1e-6)
    lvl = jnp.clip(tgt, 2.0, 5.0) - 2.0

    def sel(c0, c1, c2, c3):
        return jnp.where(
            lvl < 0.5, c0, jnp.where(lvl < 1.5, c1, jnp.where(lvl < 2.5, c2, c3)))

    scale = sel(0.25, 0.125, 0.0625, 0.03125)
    dimf = sel(32.0, 16.0, 8.0, 4.0)
    basef = sel(0.0, 32768.0, 36864.0, 37376.0)
    x1s = x1 * scale
    y1s = y1 * scale
    z1s = z1 * scale
    rw = jnp.maximum(x2 * scale - x1s, 1.0) * (1.0 / _P)
    rh = jnp.maximum(y2 * scale - y1s, 1.0) * (1.0 / _P)
    rd = jnp.maximum(z2 * scale - z1s, 1.0) * (1.0 / _P)
    cell = lax.broadcasted_iota(jnp.int32, (1, _CELLS), 1)
    pxf = (cell % _P).astype(jnp.float32)
    pyf = ((cell // _P) % _P).astype(jnp.float32)
    pzf = (cell // (_P * _P)).astype(jnp.float32)
    mx = dimf - 1.0
    x = jnp.clip(x1s + (pxf + 0.5) * rw, 0.0, mx)
    y = jnp.clip(y1s + (pyf + 0.5) * rh, 0.0, mx)
    z = jnp.clip(z1s + (pzf + 0.5) * rd, 0.0, mx)
    x0 = jnp.floor(x)
    y0 = jnp.floor(y)
    z0 = jnp.floor(z)
    lx_ref[...] = x - x0
    ly_ref[...] = y - y0
    lz_ref[...] = z - z0
    idx_ref[...] = (basef + (z0 * dimf + y0) * dimf + x0
                    + lvl * 1048576.0).astype(jnp.int32)


def _compute_meta(boxes_p):
    f32 = jnp.float32
    i32 = jnp.int32
    outs = pl.pallas_call(
        _meta_body,
        out_shape=[
            jax.ShapeDtypeStruct((_NBOX, _CELLS), i32),
            jax.ShapeDtypeStruct((_NBOX, _CELLS), f32),
            jax.ShapeDtypeStruct((_NBOX, _CELLS), f32),
            jax.ShapeDtypeStruct((_NBOX, _CELLS), f32),
        ],
    )(boxes_p)
    return [o.reshape(-1) for o in outs]


def _sc_body(tbl, midx, mlz, mly, mlx, out,
             vidx, vlz, vly, vlx, vl0, vl1, vl2, vl3, vr0, vr1, vr2, vr3,
             vo0, vo1, sm0, sm1, sm2, sm3, om0, om1):
    wid = lax.axis_index("s") * 2 + lax.axis_index("c")
    e0 = wid * (_GPW * 16)
    for src, dst in ((midx, vidx), (mlz, vlz), (mly, vly), (mlx, vlx)):
        pltpu.sync_copy(src.at[pl.ds(e0, _GPW * 16)], dst)

    lane = lax.broadcasted_iota(jnp.int32, (16,), 0)
    zeros = jnp.zeros((16,), jnp.int32)
    ones = zeros + 1
    n0 = wid * _BPW
    vls = (vl0, vl1, vl2, vl3)
    vrs = (vr0, vr1, vr2, vr3)
    sms = (sm0, sm1, sm2, sm3)
    vos = (vo0, vo1)
    oms = (om0, om1)
    dn = lax.GatherDimensionNumbers(
        offset_dims=(), collapsed_slice_dims=(0,), start_index_map=(0,))

    def fire(gl, s):
        gv = zeros + gl * 16 + lane
        ivp = plsc.load_gather(vidx, [gv])
        lvl = lax.shift_right_logical(ivp, 20)
        iv = ivp - lax.shift_left(lvl, 20)
        expw = 5 - lvl
        wv = lax.shift_left(ones, expw)
        hwv = lax.shift_left(ones, expw + expw)
        k = 0
        for az in range(2):
            for ay in range(2):
                for ax in range(2):
                    ik = iv
                    if az:
                        ik = ik + hwv
                    if ay:
                        ik = ik + wv
                    if ax:
                        ik = ik + 1
                    vls[s][pl.ds(k * 16, 16)] = ik
                    k += 1
        pltpu.async_copy(tbl.at[vls[s]], vrs[s], sms[s])

    fire(jnp.int32(0), 0)
    fire(jnp.int32(1), 1)
    fire(jnp.int32(2), 2)

    def box_loop(bi, carry):
        nb0 = n0 + bi * 2
        g0 = bi * (2 * _GPB)

        def quad_loop(it, carry2):
            for half in range(4):
                s = half
                sub2 = it * 4 + half
                g = g0 + sub2
                n = nb0 + jnp.where(sub2 >= _GPB, 1, 0)
                sub = jnp.where(sub2 >= _GPB, sub2 - _GPB, sub2)
                fire(jnp.minimum(g + 3, _GPW - 1), (half + 3) % 4)
                pltpu.make_async_copy(tbl.at[vls[s]], vrs[s], sms[s]).wait()
                gv = zeros + g * 16 + lane
                tz = plsc.load_gather(vlz, [gv])
                ty = plsc.load_gather(vly, [gv])
                tx = plsc.load_gather(vlx, [gv])
                wz = (1.0 - tz, tz)
                wy = (1.0 - ty, ty)
                wx = (1.0 - tx, tx)
                wk = [wz[az] * wy[ay] * wx[ax]
                      for az in range(2) for ay in range(2) for ax in range(2)]
                vr = vrs[s]
                so = half % 2
                vo = vos[so]

                @pl.when(g >= 2)
                def _(vo=vo, so=so):
                    pltpu.make_async_copy(
                        vo, out.at[0, pl.ds(0, 16), :], oms[so]).wait()

                def p_loop(p, carry4, wk=wk, vr=vr, vo=vo):
                    pv = zeros + p
                    wb = [lax.gather(wk[k2], pv[:, None], dn, (1,),
                                     mode=lax.GatherScatterMode.PROMISE_IN_BOUNDS)
                          for k2 in range(8)]
                    for j in range(4):
                        aa = None
                        ab = None
                        for k2 in range(8):
                            vi = vr[k2 * 16 + p, pl.ds(j * 16, 16)]
                            v = plsc.bitcast(vi, jnp.bfloat16)
                            ua, ub = plsc.unpack(
                                v, format=plsc.PackFormat.INTERLEAVED,
                                preferred_element_type=jnp.float32)
                            if aa is None:
                                aa = ua * wb[k2]
                                ab = ub * wb[k2]
                            else:
                                aa = aa + ua * wb[k2]
                                ab = ab + ub * wb[k2]
                        vo[p, pl.ds(j * 32, 16)] = aa
                        vo[p, pl.ds(j * 32 + 16, 16)] = ab
                    return carry4

                lax.fori_loop(0, 16, p_loop, 0, unroll=2)
                off = pl.multiple_of(sub * 16, 16)
                pltpu.async_copy(vo, out.at[n, pl.ds(off, 16), :], oms[so])
            return carry2

        lax.fori_loop(0, _GPB // 2, quad_loop, 0)
        return carry

    lax.fori_loop(0, _BPW // 2, box_loop, 0)
    pltpu.make_async_copy(tbl.at[vls[0]], vrs[0], sms[0]).wait()
    pltpu.make_async_copy(tbl.at[vls[1]], vrs[1], sms[1]).wait()
    pltpu.make_async_copy(tbl.at[vls[2]], vrs[2], sms[2]).wait()
    pltpu.make_async_copy(vo0, out.at[0, pl.ds(0, 16), :], om0).wait()
    pltpu.make_async_copy(vo1, out.at[0, pl.ds(0, 16), :], om1).wait()


def _sc_align(tbl, meta):
    midx, mlz, mly, mlx = meta
    i32 = jnp.int32
    f32 = jnp.float32
    ne = _GPW * 16
    return pl.kernel(
        _sc_body,
        out_type=jax.ShapeDtypeStruct((_NBOX, _CELLS, _C), f32),
        mesh=plsc.VectorSubcoreMesh(core_axis_name="c", subcore_axis_name="s"),
        compiler_params=pltpu.CompilerParams(
            needs_layout_passes=False, use_tc_tiling_on_sc=False),
        scratch_types=[
            pltpu.VMEM((ne,), i32),
            pltpu.VMEM((ne,), f32),
            pltpu.VMEM((ne,), f32),
            pltpu.VMEM((ne,), f32),
            pltpu.VMEM((128,), i32),
            pltpu.VMEM((128,), i32),
            pltpu.VMEM((128,), i32),
            pltpu.VMEM((128,), i32),
            pltpu.VMEM((128, _C // 2), i32),
            pltpu.VMEM((128, _C // 2), i32),
            pltpu.VMEM((128, _C // 2), i32),
            pltpu.VMEM((128, _C // 2), i32),
            pltpu.VMEM((16, _C), f32),
            pltpu.VMEM((16, _C), f32),
            pltpu.SemaphoreType.DMA,
            pltpu.SemaphoreType.DMA,
            pltpu.SemaphoreType.DMA,
            pltpu.SemaphoreType.DMA,
            pltpu.SemaphoreType.DMA,
            pltpu.SemaphoreType.DMA,
        ],
    )(tbl, midx, mlz, mly, mlx)


_CH_PERM = tuple(
    (p % 2) * 16 + (p % 32) // 2 + (p // 32) * 32 for p in range(_C))


def kernel(feat0, feat1, feat2, feat3, boxes):
    tbl = jnp.concatenate(
        [f[0].reshape(_C, -1).T for f in (feat0, feat1, feat2, feat3)], axis=0)
    tbl = jnp.pad(tbl, ((0, _TBL_ROWS - tbl.shape[0]), (0, 0)))
    tbl = tbl[:, jnp.array(_CH_PERM, dtype=jnp.int32)].astype(jnp.bfloat16)
    tbl = lax.bitcast_convert_type(
        tbl.reshape(_TBL_ROWS, _C // 2, 2), jnp.int32)
    nb = boxes.shape[0]
    boxes_p = jnp.pad(boxes, ((0, _NBOX - nb), (0, 0)))
    meta = _compute_meta(boxes_p)
    out = _sc_align(tbl, meta)
    return out[:nb, :343, :].transpose(0, 2, 1).reshape(nb, _C, _P, _P, _P)
